# Initial kernel scaffold; baseline (speedup 1.0000x reference)
import jax
import jax.numpy as jnp
from jax.experimental import pallas as pl


def kernel(params, nid, edge_index_r0, edge_index_r1, graph_ids):
    raise NotImplementedError("write your pallas kernel here")


# speedup vs baseline: 40.8323x; 40.8323x over previous
import functools

import jax
import jax.numpy as jnp
from jax import lax
from jax.experimental import pallas as pl
from jax.experimental.pallas import tpu as pltpu
from jax.experimental.pallas import tpu_sc as plsc

N_NODES = 50000
NUM_EMB = 500000
IN, HID, OUT, H, DH = 32, 64, 32, 4, 16
E_EDGES = 800000
B_GRAPHS = 128

NW = 32
CH = 128
NP = 53248
EP = 802816
NBLK = 256
NGRID = NP // NBLK
SROWS = NP // 16

_mesh = plsc.VectorSubcoreMesh(core_axis_name="c", subcore_axis_name="s")
_sc_params = pltpu.CompilerParams(use_tc_tiling_on_sc=False,
                                  needs_layout_passes=False)


def _embed_gather(table, idx):
    n_per_w = NP // NW
    n_ch = n_per_w // CH

    @functools.partial(
        pl.kernel,
        mesh=_mesh,
        compiler_params=_sc_params,
        out_type=jax.ShapeDtypeStruct((NP, IN), jnp.float32),
        scratch_types=[
            pltpu.VMEM((CH,), jnp.int32),
            pltpu.VMEM((CH, IN), jnp.float32),
            pltpu.SemaphoreType.DMA,
        ],
    )
    def k(tab_h, idx_h, out_h, idx_v, rows_v, sem):
        w = lax.axis_index("c") * 16 + lax.axis_index("s")

        @pl.loop(0, n_ch)
        def _(j):
            base = w * n_per_w + j * CH
            pltpu.sync_copy(idx_h.at[pl.ds(base, CH)], idx_v)
            pltpu.async_copy(tab_h.at[idx_v], rows_v, sem).wait()
            pltpu.sync_copy(rows_v, out_h.at[pl.ds(base, CH)])

    return k(table, idx)


def _full(shape):
    return pl.BlockSpec(shape, lambda i: tuple(0 for _ in shape))


def _tc_layer0(h0, w0, w1, ae0, ae1):
    din = h0.shape[1]

    def body(h_ref, w0_ref, w1_ref, a0_ref, a1_ref,
             f0_ref, f1_ref, t0_ref, t1_ref):
        hb = h_ref[...]
        for w_ref, a_ref, f_ref, t_ref in (
            (w0_ref, a0_ref, f0_ref, t0_ref),
            (w1_ref, a1_ref, f1_ref, t1_ref),
        ):
            feat = jnp.dot(hb, w_ref[...], preferred_element_type=jnp.float32)
            elr = jnp.dot(feat, a_ref[...], preferred_element_type=jnp.float32)
            t_ref[...] = jnp.concatenate(
                [elr, jnp.zeros((NBLK, 8), jnp.float32)], axis=1)
            f_ref[0] = feat[:, :32]
            f_ref[1] = feat[:, 32:]

    return pl.pallas_call(
        body,
        grid=(NGRID,),
        in_specs=[
            pl.BlockSpec((NBLK, din), lambda i: (i, 0)),
            _full((din, HID)), _full((din, HID)),
            _full((HID, 8)), _full((HID, 8)),
        ],
        out_specs=[
            pl.BlockSpec((2, NBLK, 32), lambda i: (0, i, 0)),
            pl.BlockSpec((2, NBLK, 32), lambda i: (0, i, 0)),
            pl.BlockSpec((NBLK, 16), lambda i: (i, 0)),
            pl.BlockSpec((NBLK, 16), lambda i: (i, 0)),
        ],
        out_shape=[
            jax.ShapeDtypeStruct((2, NP, 32), jnp.float32),
            jax.ShapeDtypeStruct((2, NP, 32), jnp.float32),
            jax.ShapeDtypeStruct((NP, 16), jnp.float32),
            jax.ShapeDtypeStruct((NP, 16), jnp.float32),
        ],
    )(h0, w0, w1, ae0, ae1)


def _normalize(u_lo, u_hi, sa, sb, rep_ref):
    s = sa + sb
    sinv = (1.0 / (s + 1e-9)) @ rep_ref
    return jnp.concatenate([u_lo, u_hi], axis=1) * sinv


def _tc_layer(u0, u1, s0p, s1p, rep, bsum, w0, w1, ae0, ae1):
    def body(u0a_ref, u0b_ref, u1a_ref, u1b_ref,
             s0a_ref, s0b_ref, s1a_ref, s1b_ref,
             rep_ref, b_ref, w0_ref, w1_ref, a0_ref, a1_ref,
             f0_ref, f1_ref, t0_ref, t1_ref):
        rp = rep_ref[...]
        hb = (_normalize(u0a_ref[...], u0b_ref[...], s0a_ref[...], s0b_ref[...], rp)
              + _normalize(u1a_ref[...], u1b_ref[...], s1a_ref[...], s1b_ref[...], rp)
              + b_ref[...])
        hb = jnp.maximum(hb, 0.0)
        for w_ref, a_ref, f_ref, t_ref in (
            (w0_ref, a0_ref, f0_ref, t0_ref),
            (w1_ref, a1_ref, f1_ref, t1_ref),
        ):
            feat = jnp.dot(hb, w_ref[...], preferred_element_type=jnp.float32)
            elr = jnp.dot(feat, a_ref[...], preferred_element_type=jnp.float32)
            t_ref[...] = jnp.concatenate(
                [elr, jnp.zeros((NBLK, 8), jnp.float32)], axis=1)
            f_ref[0] = feat[:, :32]
            f_ref[1] = feat[:, 32:]

    ub = lambda off: pl.BlockSpec((NBLK, 32), lambda i, off=off: (i + off, 0))
    sb_ = lambda off: pl.BlockSpec((NBLK, 16), lambda i, off=off: (i + off, 0))
    return pl.pallas_call(
        body,
        grid=(NGRID,),
        in_specs=[
            ub(0), ub(NGRID), ub(0), ub(NGRID),
            sb_(0), sb_(NGRID), sb_(0), sb_(NGRID),
            _full((16, HID)), _full((1, HID)),
            _full((HID, HID)), _full((HID, HID)),
            _full((HID, 8)), _full((HID, 8)),
        ],
        out_specs=[
            pl.BlockSpec((2, NBLK, 32), lambda i: (0, i, 0)),
            pl.BlockSpec((2, NBLK, 32), lambda i: (0, i, 0)),
            pl.BlockSpec((NBLK, 16), lambda i: (i, 0)),
            pl.BlockSpec((NBLK, 16), lambda i: (i, 0)),
        ],
        out_shape=[
            jax.ShapeDtypeStruct((2, NP, 32), jnp.float32),
            jax.ShapeDtypeStruct((2, NP, 32), jnp.float32),
            jax.ShapeDtypeStruct((NP, 16), jnp.float32),
            jax.ShapeDtypeStruct((NP, 16), jnp.float32),
        ],
    )(u0, u0, u1, u1, s0p, s0p, s1p, s1p, rep, bsum, w0, w1, ae0, ae1)


def _sc_pass1(t0, t1, srcs, dsts):
    e_w = EP // NW
    nch = e_w // CH

    @functools.partial(
        pl.kernel,
        mesh=_mesh,
        compiler_params=_sc_params,
        out_type=[
            jax.ShapeDtypeStruct((2 * EP, 4), jnp.float32),
            jax.ShapeDtypeStruct((2 * NP, 16), jnp.float32),
            jax.ShapeDtypeStruct((2 * NP, 16), jnp.float32),
        ],
        scratch_types=[
            pltpu.VMEM((CH,), jnp.int32),
            pltpu.VMEM((CH,), jnp.int32),
            pltpu.VMEM((CH, 16), jnp.float32),
            pltpu.VMEM((CH, 16), jnp.float32),
            pltpu.VMEM((CH, 4), jnp.float32),
            pltpu.VMEM((CH, 16), jnp.float32),
            pltpu.VMEM((CH, 16), jnp.float32),
            pltpu.VMEM_SHARED((NP, 16), jnp.float32),
            pltpu.VMEM_SHARED((NP, 16), jnp.float32),
            pltpu.SemaphoreType.DMA,
        ],
    )
    def k(t0_h, t1_h, srcs_h, dsts_h, exf_h, s0p_h, s1p_h,
          srcv, dstv, ts, td, exc, exs, zb, s0_sh, s1_sh, sem):
        c = lax.axis_index("c")
        t = lax.axis_index("s")
        w = c * 16 + t
        z16 = jnp.zeros((16,), jnp.float32)

        @pl.loop(0, CH)
        def _(i):
            zb[i, :] = z16
            exs[i, :] = z16

        @pl.loop(0, SROWS // CH)
        def _(j):
            r = t * SROWS + j * CH
            pltpu.sync_copy(zb, s0_sh.at[pl.ds(r, CH)])
            pltpu.sync_copy(zb, s1_sh.at[pl.ds(r, CH)])

        plsc.subcore_barrier()

        iota16 = lax.broadcasted_iota(jnp.int32, (16,), 0)
        for et, t_h, s_sh in ((0, t0_h, s0_sh), (1, t1_h, s1_sh)):
            ebase0 = et * EP + w * e_w

            @pl.loop(0, nch)
            def _(kk, ebase0=ebase0, t_h=t_h, s_sh=s_sh):
                base = ebase0 + kk * CH
                pltpu.sync_copy(srcs_h.at[pl.ds(base, CH)], srcv)
                pltpu.sync_copy(dsts_h.at[pl.ds(base, CH)], dstv)
                pltpu.async_copy(t_h.at[srcv], ts, sem).wait()
                pltpu.async_copy(t_h.at[dstv], td, sem).wait()
                for g in range(8):
                    ids = iota16 + g * 16
                    for hh in range(4):
                        fh = jnp.full((16,), hh, jnp.int32)
                        il = plsc.load_gather(ts, [ids, fh])
                        ir = plsc.load_gather(
                            td, [ids, jnp.full((16,), hh + 4, jnp.int32)])
                        v = il + ir
                        ev = jnp.exp(jnp.maximum(v, 0.2 * v))
                        plsc.store_scatter(exc, [ids, fh], ev)
                        plsc.store_scatter(exs, [ids, fh], ev)
                pltpu.sync_copy(exc, exf_h.at[pl.ds(base, CH)])
                pltpu.sync_copy(exs, s_sh.at[dstv], add=True)

        plsc.subcore_barrier()

        @pl.loop(0, SROWS // CH)
        def _(j):
            r = t * SROWS + j * CH
            o = c * NP + r
            pltpu.sync_copy(s0_sh.at[pl.ds(r, CH)], exs)
            pltpu.sync_copy(exs, s0p_h.at[pl.ds(o, CH)])
            pltpu.sync_copy(s1_sh.at[pl.ds(r, CH)], exs)
            pltpu.sync_copy(exs, s1p_h.at[pl.ds(o, CH)])

    return k(t0, t1, srcs, dsts)


def _sc_pass2(fstack, srcs, dsts, exf, et):
    e_s = EP // 16
    nch = e_s // CH

    @functools.partial(
        pl.kernel,
        mesh=_mesh,
        compiler_params=_sc_params,
        out_type=jax.ShapeDtypeStruct((2 * NP, 32), jnp.float32),
        scratch_types=[
            pltpu.VMEM((CH,), jnp.int32),
            pltpu.VMEM((CH,), jnp.int32),
            pltpu.VMEM((CH,), jnp.int32),
            pltpu.VMEM((CH, 4), jnp.float32),
            pltpu.VMEM((CH, 32), jnp.float32),
            pltpu.VMEM((CH, 32), jnp.float32),
            pltpu.VMEM((CH, 32), jnp.float32),
            pltpu.VMEM_SHARED((NP, 32), jnp.float32),
            pltpu.SemaphoreType.DMA,
        ],
    )
    def k(f_h, srcs_h, dsts_h, exf_h, u_h,
          srcv, dstv, srcv2, exr, frows, scaled, zb, u_sh, sem):
        c = lax.axis_index("c")
        t = lax.axis_index("s")
        z16 = jnp.zeros((16,), jnp.float32)

        @pl.loop(0, CH)
        def _(i):
            zb[i, 0:16] = z16
            zb[i, 16:32] = z16

        @pl.loop(0, SROWS // CH)
        def _(j):
            pltpu.sync_copy(zb, u_sh.at[pl.ds(t * SROWS + j * CH, CH)])

        plsc.subcore_barrier()

        coff = c * NP
        h0 = 2 * c
        h1 = 2 * c + 1

        @pl.loop(0, nch)
        def _(kk):
            base = et * EP + t * e_s + kk * CH
            pltpu.sync_copy(srcs_h.at[pl.ds(base, CH)], srcv)
            pltpu.sync_copy(dsts_h.at[pl.ds(base, CH)], dstv)
            pltpu.sync_copy(exf_h.at[pl.ds(base, CH)], exr)
            for g in range(8):
                sl = pl.ds(g * 16, 16)
                srcv2[sl] = srcv[sl] + coff
            pltpu.async_copy(f_h.at[srcv2], frows, sem).wait()

            @pl.loop(0, CH)
            def _(e):
                fe = jnp.full((16,), e, jnp.int32)
                g0 = plsc.load_gather(exr, [fe, jnp.full((16,), h0, jnp.int32)])
                g1 = plsc.load_gather(exr, [fe, jnp.full((16,), h1, jnp.int32)])
                scaled[e, 0:16] = frows[e, 0:16] * g0
                scaled[e, 16:32] = frows[e, 16:32] * g1

            pltpu.sync_copy(scaled, u_sh.at[dstv], add=True)

        plsc.subcore_barrier()

        @pl.loop(0, SROWS // CH)
        def _(j):
            r = t * SROWS + j * CH
            pltpu.sync_copy(u_sh.at[pl.ds(r, CH)], scaled)
            pltpu.sync_copy(scaled, u_h.at[pl.ds(coff + r, CH)])

    return k(fstack, srcs, dsts, exf)


def _tc_final(u0, u1, s0p, s1p, rep, bsum, gid3, w1, b1, w2p, b2p):
    def body(u0a_ref, u0b_ref, u1a_ref, u1b_ref,
             s0a_ref, s0b_ref, s1a_ref, s1b_ref,
             rep_ref, b_ref, g_ref, w1_ref, b1_ref, w2_ref, b2_ref,
             emb_ref, sc_ref, acc_ref):
        i = pl.program_id(0)
        rp = rep_ref[...]
        hb = (_normalize(u0a_ref[...], u0b_ref[...], s0a_ref[...], s0b_ref[...], rp)
              + _normalize(u1a_ref[...], u1b_ref[...], s1a_ref[...], s1b_ref[...], rp)
              + b_ref[...])
        gid = g_ref[0, 0, :]
        onehot = (gid[None, :] ==
                  lax.broadcasted_iota(jnp.int32, (B_GRAPHS, NBLK), 0)
                  ).astype(jnp.float32)
        part = jnp.dot(onehot, hb, preferred_element_type=jnp.float32)

        @pl.when(i == 0)
        def _():
            acc_ref[...] = jnp.zeros_like(acc_ref)

        acc_ref[...] += part

        @pl.when(i == NGRID - 1)
        def _():
            acc = acc_ref[...]
            emb_ref[...] = acc
            hid1 = jnp.dot(acc, w1_ref[...],
                           preferred_element_type=jnp.float32) + b1_ref[...]
            sc_ref[...] = jnp.dot(hid1, w2_ref[...],
                                  preferred_element_type=jnp.float32) + b2_ref[...]

    ub = lambda off: pl.BlockSpec((NBLK, 32), lambda i, off=off: (i + off, 0))
    sb_ = lambda off: pl.BlockSpec((NBLK, 16), lambda i, off=off: (i + off, 0))
    return pl.pallas_call(
        body,
        grid=(NGRID,),
        in_specs=[
            ub(0), ub(NGRID), ub(0), ub(NGRID),
            sb_(0), sb_(NGRID), sb_(0), sb_(NGRID),
            _full((16, HID)), _full((1, HID)),
            pl.BlockSpec((1, 1, NBLK), lambda i: (i, 0, 0)),
            _full((HID, OUT)), _full((1, OUT)),
            _full((OUT, 8)), _full((1, 8)),
        ],
        out_specs=[
            pl.BlockSpec((B_GRAPHS, HID), lambda i: (0, 0)),
            pl.BlockSpec((B_GRAPHS, 8), lambda i: (0, 0)),
        ],
        out_shape=[
            jax.ShapeDtypeStruct((B_GRAPHS, HID), jnp.float32),
            jax.ShapeDtypeStruct((B_GRAPHS, 8), jnp.float32),
        ],
        scratch_shapes=[pltpu.VMEM((B_GRAPHS, HID), jnp.float32)],
    )(u0, u0, u1, u1, s0p, s0p, s1p, s1p, rep, bsum, gid3, w1, b1, w2p, b2p)


def _attn_mat(p):
    eye = jnp.eye(4, dtype=jnp.float32)
    ml = p["al"][:, :, None] * eye[:, None, :]
    mr = p["ar"][:, :, None] * eye[:, None, :]
    return jnp.concatenate([ml, mr], axis=2).reshape(HID, 8)


def kernel(params, nid, edge_index_r0, edge_index_r1, graph_ids):
    f32 = jnp.float32
    nid_p = jnp.concatenate(
        [nid, jnp.zeros((NP - N_NODES,), jnp.int32)])
    pad_e = jnp.full((EP - E_EDGES,), NP - 1, jnp.int32)
    srcs = jnp.concatenate([edge_index_r0[0], pad_e,
                            edge_index_r1[0], pad_e])
    dsts = jnp.concatenate([edge_index_r0[1], pad_e,
                            edge_index_r1[1], pad_e])
    gid3 = jnp.concatenate(
        [graph_ids, jnp.full((NP - N_NODES,), B_GRAPHS, jnp.int32)]
    ).reshape(NGRID, 1, NBLK)

    rep = jnp.concatenate(
        [jnp.repeat(jnp.eye(4, dtype=f32), DH, axis=1),
         jnp.zeros((12, HID), f32)], axis=0)

    layers = params["layers"]
    h0 = _embed_gather(params["embed"], nid_p)

    u0 = u1 = s0p = s1p = None
    for l in range(3):
        p0, p1 = layers[l]["r0"], layers[l]["r1"]
        ae0, ae1 = _attn_mat(p0), _attn_mat(p1)
        if l == 0:
            f0, f1, t0, t1 = _tc_layer0(h0, p0["W"], p1["W"], ae0, ae1)
        else:
            bsum = (p_prev0["b"] + p_prev1["b"]).reshape(1, HID)
            f0, f1, t0, t1 = _tc_layer(
                u0, u1, s0p, s1p, rep, bsum, p0["W"], p1["W"], ae0, ae1)
        exf, s0p, s1p = _sc_pass1(t0, t1, srcs, dsts)
        u0 = _sc_pass2(f0.reshape(2 * NP, 32), srcs, dsts, exf, 0)
        u1 = _sc_pass2(f1.reshape(2 * NP, 32), srcs, dsts, exf, 1)
        p_prev0, p_prev1 = p0, p1

    bsum = (p_prev0["b"] + p_prev1["b"]).reshape(1, HID)
    w2p = jnp.concatenate(
        [params["W2"], jnp.zeros((OUT, 7), f32)], axis=1)
    b2p = jnp.concatenate(
        [params["b2"], jnp.zeros((7,), f32)]).reshape(1, 8)
    emb, sc = _tc_final(u0, u1, s0p, s1p, rep, bsum, gid3,
                        params["W1"], params["b1"].reshape(1, OUT), w2p, b2p)
    return emb, sc[:, 0]


# speedup vs baseline: 69.5597x; 1.7035x over previous
import functools

import jax
import jax.numpy as jnp
from jax import lax
from jax.experimental import pallas as pl
from jax.experimental.pallas import tpu as pltpu
from jax.experimental.pallas import tpu_sc as plsc

N_NODES = 50000
NUM_EMB = 500000
IN, HID, OUT, H, DH = 32, 64, 32, 4, 16
E_EDGES = 800000
B_GRAPHS = 128

NW = 32
CH = 128
NP = 53248
EP = 802816
NBLK = 256
NGRID = NP // NBLK
SROWS = NP // 16

_mesh = plsc.VectorSubcoreMesh(core_axis_name="c", subcore_axis_name="s")
_sc_params = pltpu.CompilerParams(use_tc_tiling_on_sc=False,
                                  needs_layout_passes=False)


def _embed_gather(table, idx):
    n_per_w = NP // NW
    n_ch = n_per_w // CH

    @functools.partial(
        pl.kernel,
        mesh=_mesh,
        compiler_params=_sc_params,
        out_type=jax.ShapeDtypeStruct((NP, IN), jnp.float32),
        scratch_types=[
            pltpu.VMEM((CH,), jnp.int32),
            pltpu.VMEM((CH, IN), jnp.float32),
            pltpu.SemaphoreType.DMA,
        ],
    )
    def k(tab_h, idx_h, out_h, idx_v, rows_v, sem):
        w = lax.axis_index("c") * 16 + lax.axis_index("s")

        @pl.loop(0, n_ch)
        def _(j):
            base = w * n_per_w + j * CH
            pltpu.sync_copy(idx_h.at[pl.ds(base, CH)], idx_v)
            pltpu.async_copy(tab_h.at[idx_v], rows_v, sem).wait()
            pltpu.sync_copy(rows_v, out_h.at[pl.ds(base, CH)])

    return k(table, idx)


def _full(shape):
    return pl.BlockSpec(shape, lambda i: tuple(0 for _ in shape))


def _tc_layer0(h0, w0, w1, ae0, ae1):
    din = h0.shape[1]

    def body(h_ref, w0_ref, w1_ref, a0_ref, a1_ref,
             f0_ref, f1_ref, t0_ref, t1_ref):
        hb = h_ref[...]
        for w_ref, a_ref, f_ref, t_ref in (
            (w0_ref, a0_ref, f0_ref, t0_ref),
            (w1_ref, a1_ref, f1_ref, t1_ref),
        ):
            feat = jnp.dot(hb, w_ref[...], preferred_element_type=jnp.float32)
            elr = jnp.dot(feat, a_ref[...], preferred_element_type=jnp.float32, precision=lax.Precision.HIGHEST)
            t_ref[...] = jnp.concatenate(
                [elr, jnp.zeros((NBLK, 8), jnp.float32)], axis=1)
            f_ref[0] = feat[:, :32]
            f_ref[1] = feat[:, 32:]

    return pl.pallas_call(
        body,
        grid=(NGRID,),
        in_specs=[
            pl.BlockSpec((NBLK, din), lambda i: (i, 0)),
            _full((din, HID)), _full((din, HID)),
            _full((HID, 8)), _full((HID, 8)),
        ],
        out_specs=[
            pl.BlockSpec((2, NBLK, 32), lambda i: (0, i, 0)),
            pl.BlockSpec((2, NBLK, 32), lambda i: (0, i, 0)),
            pl.BlockSpec((NBLK, 16), lambda i: (i, 0)),
            pl.BlockSpec((NBLK, 16), lambda i: (i, 0)),
        ],
        out_shape=[
            jax.ShapeDtypeStruct((2, NP, 32), jnp.float32),
            jax.ShapeDtypeStruct((2, NP, 32), jnp.float32),
            jax.ShapeDtypeStruct((NP, 16), jnp.float32),
            jax.ShapeDtypeStruct((NP, 16), jnp.float32),
        ],
    )(h0, w0, w1, ae0, ae1)


def _normalize(u_lo, u_hi, sa, sb, rep_ref):
    s = sa + sb
    sinv = jnp.dot(1.0 / (s + 1e-9), rep_ref, precision=lax.Precision.HIGHEST)
    return jnp.concatenate([u_lo, u_hi], axis=1) * sinv


def _tc_layer(u0, u1, s0p, s1p, rep, bsum, w0, w1, ae0, ae1):
    def body(u0a_ref, u0b_ref, u1a_ref, u1b_ref,
             s0a_ref, s0b_ref, s1a_ref, s1b_ref,
             rep_ref, b_ref, w0_ref, w1_ref, a0_ref, a1_ref,
             f0_ref, f1_ref, t0_ref, t1_ref):
        rp = rep_ref[...]
        hb = (_normalize(u0a_ref[...], u0b_ref[...], s0a_ref[...], s0b_ref[...], rp)
              + _normalize(u1a_ref[...], u1b_ref[...], s1a_ref[...], s1b_ref[...], rp)
              + b_ref[...])
        hb = jnp.maximum(hb, 0.0)
        for w_ref, a_ref, f_ref, t_ref in (
            (w0_ref, a0_ref, f0_ref, t0_ref),
            (w1_ref, a1_ref, f1_ref, t1_ref),
        ):
            feat = jnp.dot(hb, w_ref[...], preferred_element_type=jnp.float32)
            elr = jnp.dot(feat, a_ref[...], preferred_element_type=jnp.float32, precision=lax.Precision.HIGHEST)
            t_ref[...] = jnp.concatenate(
                [elr, jnp.zeros((NBLK, 8), jnp.float32)], axis=1)
            f_ref[0] = feat[:, :32]
            f_ref[1] = feat[:, 32:]

    ub = lambda off: pl.BlockSpec((NBLK, 32), lambda i, off=off: (i + off, 0))
    sb_ = lambda off: pl.BlockSpec((NBLK, 16), lambda i, off=off: (i + off, 0))
    return pl.pallas_call(
        body,
        grid=(NGRID,),
        in_specs=[
            ub(0), ub(NGRID), ub(0), ub(NGRID),
            sb_(0), sb_(NGRID), sb_(0), sb_(NGRID),
            _full((16, HID)), _full((1, HID)),
            _full((HID, HID)), _full((HID, HID)),
            _full((HID, 8)), _full((HID, 8)),
        ],
        out_specs=[
            pl.BlockSpec((2, NBLK, 32), lambda i: (0, i, 0)),
            pl.BlockSpec((2, NBLK, 32), lambda i: (0, i, 0)),
            pl.BlockSpec((NBLK, 16), lambda i: (i, 0)),
            pl.BlockSpec((NBLK, 16), lambda i: (i, 0)),
        ],
        out_shape=[
            jax.ShapeDtypeStruct((2, NP, 32), jnp.float32),
            jax.ShapeDtypeStruct((2, NP, 32), jnp.float32),
            jax.ShapeDtypeStruct((NP, 16), jnp.float32),
            jax.ShapeDtypeStruct((NP, 16), jnp.float32),
        ],
    )(u0, u0, u1, u1, s0p, s0p, s1p, s1p, rep, bsum, w0, w1, ae0, ae1)


def _sc_pass1(t0, t1, srcs, dsts):
    e_w = EP // NW
    nch = e_w // CH

    @functools.partial(
        pl.kernel,
        mesh=_mesh,
        compiler_params=_sc_params,
        out_type=[
            jax.ShapeDtypeStruct((2 * EP, 4), jnp.float32),
            jax.ShapeDtypeStruct((2 * NP, 16), jnp.float32),
            jax.ShapeDtypeStruct((2 * NP, 16), jnp.float32),
        ],
        scratch_types=[
            pltpu.VMEM((2, CH), jnp.int32),
            pltpu.VMEM((2, CH), jnp.int32),
            pltpu.VMEM((2, CH, 16), jnp.float32),
            pltpu.VMEM((2, CH, 16), jnp.float32),
            pltpu.VMEM((CH, 4), jnp.float32),
            pltpu.VMEM((CH, 16), jnp.float32),
            pltpu.VMEM((CH, 16), jnp.float32),
            pltpu.VMEM_SHARED((NP, 16), jnp.float32),
            pltpu.VMEM_SHARED((NP, 16), jnp.float32),
            pltpu.SemaphoreType.DMA,
            pltpu.SemaphoreType.DMA,
            pltpu.SemaphoreType.DMA,
            pltpu.SemaphoreType.DMA,
        ],
    )
    def k(t0_h, t1_h, srcs_h, dsts_h, exf_h, s0p_h, s1p_h,
          srcv, dstv, ts, td, exc, exs, zb, s0_sh, s1_sh,
          seml0, seml1, semg0, semg1):
        c = lax.axis_index("c")
        t = lax.axis_index("s")
        w = c * 16 + t
        seml = (seml0, seml1)
        semg = (semg0, semg1)
        z16 = jnp.zeros((16,), jnp.float32)

        @pl.loop(0, CH)
        def _(i):
            zb[i, :] = z16
            exs[i, :] = z16

        @pl.loop(0, SROWS // CH)
        def _(j):
            r = t * SROWS + j * CH
            pltpu.sync_copy(zb, s0_sh.at[pl.ds(r, CH)])
            pltpu.sync_copy(zb, s1_sh.at[pl.ds(r, CH)])

        plsc.subcore_barrier()

        iota16 = lax.broadcasted_iota(jnp.int32, (16,), 0)
        for et, t_h, s_sh in ((0, t0_h, s0_sh), (1, t1_h, s1_sh)):
            ebase0 = et * EP + w * e_w

            def lin_issue(g, b, ebase0=ebase0):
                bs = ebase0 + g * CH
                pltpu.async_copy(srcs_h.at[pl.ds(bs, CH)], srcv.at[b], seml[b])
                pltpu.async_copy(dsts_h.at[pl.ds(bs, CH)], dstv.at[b], seml[b])

            def lin_wait(g, b, ebase0=ebase0):
                bs = ebase0 + g * CH
                pltpu.make_async_copy(
                    srcs_h.at[pl.ds(bs, CH)], srcv.at[b], seml[b]).wait()
                pltpu.make_async_copy(
                    dsts_h.at[pl.ds(bs, CH)], dstv.at[b], seml[b]).wait()

            def gat_issue(b, t_h=t_h):
                pltpu.async_copy(t_h.at[srcv.at[b]], ts.at[b], semg[b])
                pltpu.async_copy(t_h.at[dstv.at[b]], td.at[b], semg[b])

            def gat_wait(b, t_h=t_h):
                pltpu.make_async_copy(t_h.at[srcv.at[b]], ts.at[b], semg[b]).wait()
                pltpu.make_async_copy(t_h.at[dstv.at[b]], td.at[b], semg[b]).wait()

            def compute(b, ebase0=ebase0, s_sh=s_sh):
                def body(kk):
                    fb = jnp.full((16,), b, jnp.int32)
                    for g in range(8):
                        ids = iota16 + g * 16
                        for hh in range(4):
                            fh = jnp.full((16,), hh, jnp.int32)
                            il = plsc.load_gather(ts, [fb, ids, fh])
                            ir = plsc.load_gather(
                                td, [fb, ids, jnp.full((16,), hh + 4, jnp.int32)])
                            v = il + ir
                            ev = jnp.exp(jnp.maximum(v, 0.2 * v))
                            plsc.store_scatter(exc, [ids, fh], ev)
                            plsc.store_scatter(exs, [ids, fh], ev)
                    pltpu.sync_copy(exc, exf_h.at[pl.ds(ebase0 + kk * CH, CH)])
                    pltpu.sync_copy(exs, s_sh.at[dstv.at[b]], add=True)
                return body

            lin_issue(0, 0)
            lin_issue(1, 1)
            lin_wait(0, 0)
            gat_issue(0)

            @pl.loop(0, nch // 2)
            def _(gg, lin_issue=lin_issue, lin_wait=lin_wait,
                  gat_issue=gat_issue, gat_wait=gat_wait, compute=compute):
                for j in range(2):
                    kk = gg * 2 + j
                    b = j
                    bn = 1 - j

                    @pl.when(kk + 1 < nch)
                    def _():
                        lin_wait(kk + 1, bn)
                        gat_issue(bn)

                    gat_wait(b)
                    compute(b)(kk)

                    @pl.when(kk + 2 < nch)
                    def _():
                        lin_issue(kk + 2, b)

        plsc.subcore_barrier()

        @pl.loop(0, SROWS // CH)
        def _(j):
            r = t * SROWS + j * CH
            o = c * NP + r
            pltpu.sync_copy(s0_sh.at[pl.ds(r, CH)], exs)
            pltpu.sync_copy(exs, s0p_h.at[pl.ds(o, CH)])
            pltpu.sync_copy(s1_sh.at[pl.ds(r, CH)], exs)
            pltpu.sync_copy(exs, s1p_h.at[pl.ds(o, CH)])

    return k(t0, t1, srcs, dsts)


def _sc_pass2(fstack, srcs, dsts, exf, et):
    e_s = EP // 16
    nch = e_s // CH

    @functools.partial(
        pl.kernel,
        mesh=_mesh,
        compiler_params=_sc_params,
        out_type=jax.ShapeDtypeStruct((2 * NP, 32), jnp.float32),
        scratch_types=[
            pltpu.VMEM((2, CH), jnp.int32),
            pltpu.VMEM((2, CH), jnp.int32),
            pltpu.VMEM((2, CH), jnp.int32),
            pltpu.VMEM((2, CH, 4), jnp.float32),
            pltpu.VMEM((2, CH, 32), jnp.float32),
            pltpu.VMEM((CH, 32), jnp.float32),
            pltpu.VMEM((CH, 32), jnp.float32),
            pltpu.VMEM_SHARED((NP, 32), jnp.float32),
            pltpu.SemaphoreType.DMA,
            pltpu.SemaphoreType.DMA,
            pltpu.SemaphoreType.DMA,
            pltpu.SemaphoreType.DMA,
        ],
    )
    def k(f_h, srcs_h, dsts_h, exf_h, u_h,
          srcv, dstv, srcv2, exr, frows, scaled, zb, u_sh,
          seml0, seml1, semg0, semg1):
        c = lax.axis_index("c")
        t = lax.axis_index("s")
        seml = (seml0, seml1)
        semg = (semg0, semg1)
        z16 = jnp.zeros((16,), jnp.float32)

        @pl.loop(0, CH)
        def _(i):
            zb[i, 0:16] = z16
            zb[i, 16:32] = z16

        @pl.loop(0, SROWS // CH)
        def _(j):
            pltpu.sync_copy(zb, u_sh.at[pl.ds(t * SROWS + j * CH, CH)])

        plsc.subcore_barrier()

        coff = c * NP
        h0 = 2 * c
        h1 = 2 * c + 1

        def ebase(g):
            return et * EP + t * e_s + g * CH

        def lin_issue(g, b):
            bs = ebase(g)
            pltpu.async_copy(srcs_h.at[pl.ds(bs, CH)], srcv.at[b], seml[b])
            pltpu.async_copy(dsts_h.at[pl.ds(bs, CH)], dstv.at[b], seml[b])
            pltpu.async_copy(exf_h.at[pl.ds(bs, CH)], exr.at[b], seml[b])

        def lin_wait(g, b):
            bs = ebase(g)
            pltpu.make_async_copy(srcs_h.at[pl.ds(bs, CH)], srcv.at[b], seml[b]).wait()
            pltpu.make_async_copy(dsts_h.at[pl.ds(bs, CH)], dstv.at[b], seml[b]).wait()
            pltpu.make_async_copy(exf_h.at[pl.ds(bs, CH)], exr.at[b], seml[b]).wait()

        def gat_issue(b):
            for g8 in range(8):
                sl = pl.ds(g8 * 16, 16)
                srcv2[b, sl] = srcv[b, sl] + coff
            pltpu.async_copy(f_h.at[srcv2.at[b]], frows.at[b], semg[b])

        def gat_wait(b):
            pltpu.make_async_copy(f_h.at[srcv2.at[b]], frows.at[b], semg[b]).wait()

        def compute(b):
            fb = jnp.full((16,), b, jnp.int32)

            @pl.loop(0, CH, step=4)
            def _(e0):
                for j in range(4):
                    e = e0 + j
                    fe = jnp.full((16,), e, jnp.int32)
                    g0 = plsc.load_gather(
                        exr, [fb, fe, jnp.full((16,), h0, jnp.int32)])
                    g1 = plsc.load_gather(
                        exr, [fb, fe, jnp.full((16,), h1, jnp.int32)])
                    scaled[e, 0:16] = frows[b, e, 0:16] * g0
                    scaled[e, 16:32] = frows[b, e, 16:32] * g1

            pltpu.sync_copy(scaled, u_sh.at[dstv.at[b]], add=True)

        lin_issue(0, 0)
        lin_issue(1, 1)
        lin_wait(0, 0)
        gat_issue(0)

        @pl.loop(0, nch // 2)
        def _(gg):
            for j in range(2):
                kk = gg * 2 + j
                b = j
                bn = 1 - j

                @pl.when(kk + 1 < nch)
                def _():
                    lin_wait(kk + 1, bn)
                    gat_issue(bn)

                gat_wait(b)
                compute(b)

                @pl.when(kk + 2 < nch)
                def _():
                    lin_issue(kk + 2, b)

        plsc.subcore_barrier()

        @pl.loop(0, SROWS // CH)
        def _(j):
            r = t * SROWS + j * CH
            pltpu.sync_copy(u_sh.at[pl.ds(r, CH)], scaled)
            pltpu.sync_copy(scaled, u_h.at[pl.ds(coff + r, CH)])

    return k(fstack, srcs, dsts, exf)


def _tc_final(u0, u1, s0p, s1p, rep, bsum, gid3, w1, b1, w2p, b2p):
    def body(u0a_ref, u0b_ref, u1a_ref, u1b_ref,
             s0a_ref, s0b_ref, s1a_ref, s1b_ref,
             rep_ref, b_ref, g_ref, w1_ref, b1_ref, w2_ref, b2_ref,
             emb_ref, sc_ref, acc_ref):
        i = pl.program_id(0)
        rp = rep_ref[...]
        hb = (_normalize(u0a_ref[...], u0b_ref[...], s0a_ref[...], s0b_ref[...], rp)
              + _normalize(u1a_ref[...], u1b_ref[...], s1a_ref[...], s1b_ref[...], rp)
              + b_ref[...])
        gid = g_ref[0, 0, :]
        onehot = (gid[None, :] ==
                  lax.broadcasted_iota(jnp.int32, (B_GRAPHS, NBLK), 0)
                  ).astype(jnp.float32)
        part = jnp.dot(onehot, hb, preferred_element_type=jnp.float32, precision=lax.Precision.HIGHEST)

        @pl.when(i == 0)
        def _():
            acc_ref[...] = jnp.zeros_like(acc_ref)

        acc_ref[...] += part

        @pl.when(i == NGRID - 1)
        def _():
            acc = acc_ref[...]
            emb_ref[...] = acc
            hid1 = jnp.dot(acc, w1_ref[...],
                           preferred_element_type=jnp.float32) + b1_ref[...]
            sc_ref[...] = jnp.dot(hid1, w2_ref[...],
                                  preferred_element_type=jnp.float32) + b2_ref[...]

    ub = lambda off: pl.BlockSpec((NBLK, 32), lambda i, off=off: (i + off, 0))
    sb_ = lambda off: pl.BlockSpec((NBLK, 16), lambda i, off=off: (i + off, 0))
    return pl.pallas_call(
        body,
        grid=(NGRID,),
        in_specs=[
            ub(0), ub(NGRID), ub(0), ub(NGRID),
            sb_(0), sb_(NGRID), sb_(0), sb_(NGRID),
            _full((16, HID)), _full((1, HID)),
            pl.BlockSpec((1, 1, NBLK), lambda i: (i, 0, 0)),
            _full((HID, OUT)), _full((1, OUT)),
            _full((OUT, 8)), _full((1, 8)),
        ],
        out_specs=[
            pl.BlockSpec((B_GRAPHS, HID), lambda i: (0, 0)),
            pl.BlockSpec((B_GRAPHS, 8), lambda i: (0, 0)),
        ],
        out_shape=[
            jax.ShapeDtypeStruct((B_GRAPHS, HID), jnp.float32),
            jax.ShapeDtypeStruct((B_GRAPHS, 8), jnp.float32),
        ],
        scratch_shapes=[pltpu.VMEM((B_GRAPHS, HID), jnp.float32)],
    )(u0, u0, u1, u1, s0p, s0p, s1p, s1p, rep, bsum, gid3, w1, b1, w2p, b2p)


def _attn_mat(p):
    eye = jnp.eye(4, dtype=jnp.float32)
    ml = p["al"][:, :, None] * eye[:, None, :]
    mr = p["ar"][:, :, None] * eye[:, None, :]
    return jnp.concatenate([ml, mr], axis=2).reshape(HID, 8)


def kernel(params, nid, edge_index_r0, edge_index_r1, graph_ids):
    f32 = jnp.float32
    nid_p = jnp.concatenate(
        [nid, jnp.zeros((NP - N_NODES,), jnp.int32)])
    pad_e = jnp.full((EP - E_EDGES,), NP - 1, jnp.int32)
    srcs = jnp.concatenate([edge_index_r0[0], pad_e,
                            edge_index_r1[0], pad_e])
    dsts = jnp.concatenate([edge_index_r0[1], pad_e,
                            edge_index_r1[1], pad_e])
    gid3 = jnp.concatenate(
        [graph_ids, jnp.full((NP - N_NODES,), B_GRAPHS, jnp.int32)]
    ).reshape(NGRID, 1, NBLK)

    rep = jnp.concatenate(
        [jnp.repeat(jnp.eye(4, dtype=f32), DH, axis=1),
         jnp.zeros((12, HID), f32)], axis=0)

    layers = params["layers"]
    h0 = _embed_gather(params["embed"], nid_p)

    u0 = u1 = s0p = s1p = None
    for l in range(3):
        p0, p1 = layers[l]["r0"], layers[l]["r1"]
        ae0, ae1 = _attn_mat(p0), _attn_mat(p1)
        if l == 0:
            f0, f1, t0, t1 = _tc_layer0(h0, p0["W"], p1["W"], ae0, ae1)
        else:
            bsum = (p_prev0["b"] + p_prev1["b"]).reshape(1, HID)
            f0, f1, t0, t1 = _tc_layer(
                u0, u1, s0p, s1p, rep, bsum, p0["W"], p1["W"], ae0, ae1)
        exf, s0p, s1p = _sc_pass1(t0, t1, srcs, dsts)
        u0 = _sc_pass2(f0.reshape(2 * NP, 32), srcs, dsts, exf, 0)
        u1 = _sc_pass2(f1.reshape(2 * NP, 32), srcs, dsts, exf, 1)
        p_prev0, p_prev1 = p0, p1

    bsum = (p_prev0["b"] + p_prev1["b"]).reshape(1, HID)
    w2p = jnp.concatenate(
        [params["W2"], jnp.zeros((OUT, 7), f32)], axis=1)
    b2p = jnp.concatenate(
        [params["b2"], jnp.zeros((7,), f32)]).reshape(1, 8)
    emb, sc = _tc_final(u0, u1, s0p, s1p, rep, bsum, gid3,
                        params["W1"], params["b1"].reshape(1, OUT), w2p, b2p)
    return emb, sc[:, 0]
